# baseline (speedup 1.0000x reference)
import math

import jax
import jax.numpy as jnp
from jax import lax
from jax.experimental import pallas as pl
from jax.experimental.pallas import tpu as pltpu
from jax.experimental.pallas import tpu_sc as plsc

NUM_BUCKETS = 32
NUM_HEADS = 16
Q = 2048
K = 2048
NB = NUM_BUCKETS // 2
MAX_EXACT = NB // 2
MAX_DISTANCE = 128

TPAD = 4224
W8ROW = 4096
BATCH = 8
ROWS_PER_TILE = Q // 2
GROUPS = ROWS_PER_TILE // 8


def _tc_diag_body(shift_ref, table_ref, out_ref):
    shift = shift_ref[0]
    t = lax.broadcasted_iota(jnp.int32, (1, TPAD), 1)
    d = t - (Q - 1) + shift
    buckets = (d > 0).astype(jnp.int32) * NB
    rp = jnp.abs(d)
    is_small = rp < MAX_EXACT
    rp_safe = jnp.maximum(rp, 1)
    rp_if_large = MAX_EXACT + (
        jnp.log(rp_safe.astype(jnp.float32) / MAX_EXACT)
        / math.log(MAX_DISTANCE / MAX_EXACT)
        * (NB - MAX_EXACT)
    ).astype(jnp.int32)
    rp_if_large = jnp.minimum(rp_if_large, NB - 1)
    b = buckets + jnp.where(is_small, rp, rp_if_large)
    table = table_ref[...]
    w = jnp.zeros((NUM_HEADS, TPAD), jnp.float32)
    for v in range(NUM_BUCKETS):
        tcol = table[v, :].reshape(NUM_HEADS, 1)
        w = jnp.where(b == v, tcol, w)
    for k in range(8):
        out_ref[:, k, :] = w[:, (7 - k):(7 - k) + W8ROW]


def _sc_body(w8_hbm, out_hbm, w8_v, sem):
    head = lax.axis_index("s")
    half = lax.axis_index("c")

    pltpu.sync_copy(w8_hbm.at[pl.ds(head * 8, 8), :], w8_v)

    base_row = half * ROWS_PER_TILE

    def row_group(g, carry):
        cps = []
        for u in range(BATCH):
            i = base_row + (g * BATCH + u) * 8
            base = pl.multiple_of((Q - 8) - i, 8)
            cps.append(
                pltpu.async_copy(
                    w8_v.at[:, pl.ds(base, K)],
                    out_hbm.at[pl.ds(head * Q + i, 8), :],
                    sem,
                )
            )
        for cp in cps:
            cp.wait()
        return carry

    lax.fori_loop(0, GROUPS // BATCH, row_group, 0)


def kernel(query_length, key_length, table):
    shift = (jnp.asarray(key_length, jnp.int32) - K) - (
        jnp.asarray(query_length, jnp.int32) - Q
    )
    shift_arr = jnp.reshape(shift, (1,))

    w8 = pl.pallas_call(
        _tc_diag_body,
        out_shape=jax.ShapeDtypeStruct((NUM_HEADS, 8, W8ROW), jnp.float32),
        in_specs=[
            pl.BlockSpec(memory_space=pltpu.SMEM),
            pl.BlockSpec(),
        ],
    )(shift_arr, table)
    w8_2d = w8.reshape(NUM_HEADS * 8, W8ROW)

    mesh = plsc.VectorSubcoreMesh(core_axis_name="c", subcore_axis_name="s")
    out2d = pl.kernel(
        _sc_body,
        out_type=jax.ShapeDtypeStruct((NUM_HEADS * Q, K), jnp.float32),
        mesh=mesh,
        scratch_types=[
            pltpu.VMEM((8, W8ROW), jnp.float32),
            pltpu.SemaphoreType.DMA,
        ],
        compiler_params=pltpu.CompilerParams(use_tc_tiling_on_sc=False),
    )(w8_2d)
    return out2d.reshape(1, NUM_HEADS, Q, K)


# speedup vs baseline: 1.0044x; 1.0044x over previous
import math

import jax
import jax.numpy as jnp
from jax import lax
from jax.experimental import pallas as pl
from jax.experimental.pallas import tpu as pltpu
from jax.experimental.pallas import tpu_sc as plsc

NUM_BUCKETS = 32
NUM_HEADS = 16
Q = 2048
K = 2048
NB = NUM_BUCKETS // 2
MAX_EXACT = NB // 2
MAX_DISTANCE = 128

TPAD = 4224
W8ROW = 4096
BATCH = 8
ROWS_PER_TILE = Q // 2
GROUPS = ROWS_PER_TILE // 8


def _tc_diag_body(shift_ref, table_ref, out_ref):
    shift = shift_ref[0]
    t = lax.broadcasted_iota(jnp.int32, (1, TPAD), 1)
    d = t - (Q - 1) + shift
    buckets = (d > 0).astype(jnp.int32) * NB
    rp = jnp.abs(d)
    is_small = rp < MAX_EXACT
    rp_safe = jnp.maximum(rp, 1)
    rp_if_large = MAX_EXACT + (
        jnp.log(rp_safe.astype(jnp.float32) / MAX_EXACT)
        / math.log(MAX_DISTANCE / MAX_EXACT)
        * (NB - MAX_EXACT)
    ).astype(jnp.int32)
    rp_if_large = jnp.minimum(rp_if_large, NB - 1)
    b = buckets + jnp.where(is_small, rp, rp_if_large)
    table = table_ref[...]
    w = jnp.zeros((NUM_HEADS, TPAD), jnp.float32)
    for v in range(NUM_BUCKETS):
        tcol = table[v, :].reshape(NUM_HEADS, 1)
        w = jnp.where(b == v, tcol, w)
    for k in range(8):
        out_ref[:, k, :] = w[:, (7 - k):(7 - k) + W8ROW]


def _sc_body(w8_hbm, out_hbm, w8_v, sem):
    head = lax.axis_index("s")
    half = lax.axis_index("c")

    pltpu.sync_copy(w8_hbm.at[pl.ds(head * 8, 8), :], w8_v)

    base_row = half * ROWS_PER_TILE

    def row_group(g, carry):
        cps = []
        for u in range(BATCH):
            i = base_row + (g * BATCH + u) * 8
            base = pl.multiple_of((Q - 8) - i, 8)
            cps.append(
                pltpu.async_copy(
                    w8_v.at[:, pl.ds(base, K)],
                    out_hbm.at[0, head, pl.ds(i, 8), :],
                    sem,
                )
            )
        for cp in cps:
            cp.wait()
        return carry

    lax.fori_loop(0, GROUPS // BATCH, row_group, 0)


def kernel(query_length, key_length, table):
    shift = (jnp.asarray(key_length, jnp.int32) - K) - (
        jnp.asarray(query_length, jnp.int32) - Q
    )
    shift_arr = jnp.reshape(shift, (1,))

    w8 = pl.pallas_call(
        _tc_diag_body,
        out_shape=jax.ShapeDtypeStruct((NUM_HEADS, 8, W8ROW), jnp.float32),
        in_specs=[
            pl.BlockSpec(memory_space=pltpu.SMEM),
            pl.BlockSpec(),
        ],
    )(shift_arr, table)
    w8_2d = w8.reshape(NUM_HEADS * 8, W8ROW)

    mesh = plsc.VectorSubcoreMesh(core_axis_name="c", subcore_axis_name="s")
    return pl.kernel(
        _sc_body,
        out_type=jax.ShapeDtypeStruct((1, NUM_HEADS, Q, K), jnp.float32),
        mesh=mesh,
        scratch_types=[
            pltpu.VMEM((8, W8ROW), jnp.float32),
            pltpu.SemaphoreType.DMA,
        ],
        compiler_params=pltpu.CompilerParams(use_tc_tiling_on_sc=False),
    )(w8_2d)


# speedup vs baseline: 1.6439x; 1.6366x over previous
import math

import jax
import jax.numpy as jnp
from jax import lax
from jax.experimental import pallas as pl
from jax.experimental.pallas import tpu as pltpu
from jax.experimental.pallas import tpu_sc as plsc

NUM_BUCKETS = 32
NUM_HEADS = 16
Q = 2048
K = 2048
NB = NUM_BUCKETS // 2
MAX_EXACT = NB // 2
MAX_DISTANCE = 128

TPAD = 4224
SROW = 4096
NPHASE = 16
NSLOT = 4
GROUPS = (Q // 2) // 8


def _tc_stair_body(shift_ref, table_ref, out_ref):
    shift = shift_ref[0]
    t = lax.broadcasted_iota(jnp.int32, (1, TPAD), 1)
    d = t - (Q - 1) + shift
    buckets = (d > 0).astype(jnp.int32) * NB
    rp = jnp.abs(d)
    is_small = rp < MAX_EXACT
    rp_safe = jnp.maximum(rp, 1)
    rp_if_large = MAX_EXACT + (
        jnp.log(rp_safe.astype(jnp.float32) / MAX_EXACT)
        / math.log(MAX_DISTANCE / MAX_EXACT)
        * (NB - MAX_EXACT)
    ).astype(jnp.int32)
    rp_if_large = jnp.minimum(rp_if_large, NB - 1)
    b = buckets + jnp.where(is_small, rp, rp_if_large)
    table = table_ref[...]
    w = jnp.zeros((NUM_HEADS, TPAD), jnp.float32)
    for v in range(NUM_BUCKETS):
        tcol = table[v, :].reshape(NUM_HEADS, 1)
        w = jnp.where(b == v, tcol, w)
    for p in range(NPHASE):
        for k in range(8):
            start = 8 * p + (7 - k)
            out_ref[p, :, k, :] = w[:, start:start + SROW]


def _sc_body(stair_hbm, out_hbm, b0, b1, b2, b3,
             si0, si1, si2, si3, so0, so1, so2, so3):
    head = lax.axis_index("s")
    half = lax.axis_index("c")
    bufs = (b0, b1, b2, b3)
    isems = (si0, si1, si2, si3)
    osems = (so0, so1, so2, so3)

    base_row = half * (Q // 2)

    def src_ref(g):
        i = base_row + 8 * g
        t = (Q - 8) - i
        p = (t >> 3) & (NPHASE - 1)
        a = pl.multiple_of(t - 8 * p, 128)
        row = pl.multiple_of((p * NUM_HEADS + head) * 8, 8)
        return stair_hbm.at[pl.ds(row, 8), pl.ds(a, K)]

    def dst_ref(g):
        i = base_row + 8 * g
        return out_hbm.at[0, head, pl.ds(i, 8), :]

    for s in range(NSLOT):
        pltpu.async_copy(src_ref(s), bufs[s], isems[s])

    def macro(m, carry):
        for s in range(NSLOT):
            g = m * NSLOT + s
            pltpu.make_async_copy(src_ref(g), bufs[s], isems[s]).wait()
            pltpu.async_copy(bufs[s], dst_ref(g), osems[s])
        for s in range(NSLOT):
            g_next = m * NSLOT + s + NSLOT

            @pl.when(g_next < GROUPS)
            def _refill(s=s, g_next=g_next):
                pltpu.make_async_copy(bufs[s], dst_ref(0), osems[s]).wait()
                pltpu.async_copy(src_ref(g_next), bufs[s], isems[s])

        return carry

    lax.fori_loop(0, GROUPS // NSLOT, macro, 0)

    for s in range(NSLOT):
        pltpu.make_async_copy(bufs[s], dst_ref(0), osems[s]).wait()


def kernel(query_length, key_length, table):
    shift = (jnp.asarray(key_length, jnp.int32) - K) - (
        jnp.asarray(query_length, jnp.int32) - Q
    )
    shift_arr = jnp.reshape(shift, (1,))

    stair = pl.pallas_call(
        _tc_stair_body,
        out_shape=jax.ShapeDtypeStruct((NPHASE, NUM_HEADS, 8, SROW), jnp.float32),
        in_specs=[
            pl.BlockSpec(memory_space=pltpu.SMEM),
            pl.BlockSpec(),
        ],
        compiler_params=pltpu.CompilerParams(
            vmem_limit_bytes=100 * 1024 * 1024,
        ),
    )(shift_arr, table)
    stair2d = stair.reshape(NPHASE * NUM_HEADS * 8, SROW)

    mesh = plsc.VectorSubcoreMesh(core_axis_name="c", subcore_axis_name="s")
    return pl.kernel(
        _sc_body,
        out_type=jax.ShapeDtypeStruct((1, NUM_HEADS, Q, K), jnp.float32),
        mesh=mesh,
        scratch_types=(
            [pltpu.VMEM((8, K), jnp.float32)] * NSLOT
            + [pltpu.SemaphoreType.DMA] * (2 * NSLOT)
        ),
        compiler_params=pltpu.CompilerParams(use_tc_tiling_on_sc=True),
    )(stair2d)


# speedup vs baseline: 2.3886x; 1.4531x over previous
import math

import jax
import jax.numpy as jnp
from jax import lax
from jax.experimental import pallas as pl
from jax.experimental.pallas import tpu as pltpu
from jax.experimental.pallas import tpu_sc as plsc

NUM_BUCKETS = 32
NUM_HEADS = 16
Q = 2048
K = 2048
NB = NUM_BUCKETS // 2
MAX_EXACT = NB // 2
MAX_DISTANCE = 128

TPAD = 4224
SROW = 4096
NPHASE = 16
NSLOT = 4
GROUPS = (Q // 2) // 8


def _tc_stair_body(shift_ref, table_ref, out_ref):
    shift = shift_ref[0]
    t = lax.broadcasted_iota(jnp.int32, (1, TPAD), 1)
    d = t - (Q - 1) + shift
    buckets = (d > 0).astype(jnp.int32) * NB
    rp = jnp.abs(d)
    is_small = rp < MAX_EXACT
    rp_safe = jnp.maximum(rp, 1)
    rp_if_large = MAX_EXACT + (
        jnp.log(rp_safe.astype(jnp.float32) / MAX_EXACT)
        / math.log(MAX_DISTANCE / MAX_EXACT)
        * (NB - MAX_EXACT)
    ).astype(jnp.int32)
    rp_if_large = jnp.minimum(rp_if_large, NB - 1)
    b = buckets + jnp.where(is_small, rp, rp_if_large)
    table = table_ref[...]
    w = jnp.zeros((NUM_HEADS, TPAD), jnp.float32)
    for v in range(NUM_BUCKETS):
        tcol = table[v, :].reshape(NUM_HEADS, 1)
        w = jnp.where(b == v, tcol, w)
    for p in range(NPHASE):
        for k in range(8):
            start = 8 * p + (7 - k)
            out_ref[p, :, k, :] = w[:, start:start + SROW]


SLABW = 2944
JSTRIDE = 16


def _sc_body(stair_hbm, out_hbm, b0, b1, b2,
             si0, si1, si2, so0, so1, so2):
    head = lax.axis_index("s")
    half = lax.axis_index("c")
    slabs = (b0, b1, b2)
    isems = (si0, si1, si2)
    osems = (so0, so1, so2)

    base_row = half * (Q // 2)
    slab_col = pl.multiple_of((Q // 2) - base_row, 128)

    def fire_in(p, s):
        row = pl.multiple_of((p * NUM_HEADS + head) * 8, 8)
        pltpu.async_copy(
            stair_hbm.at[pl.ds(row, 8), pl.ds(slab_col, SLABW)],
            slabs[s], isems[s],
        )

    def wait_in(s):
        pltpu.make_async_copy(
            stair_hbm.at[pl.ds(0, 8), pl.ds(0, SLABW)], slabs[s], isems[s]
        ).wait()

    def dst_ref(g):
        i = base_row + 8 * g
        return out_hbm.at[0, head, pl.ds(i, 8), :]

    def fire_outs(p, s):
        gp = (NPHASE - 1) - p
        for j in range(8):
            g = gp + JSTRIDE * j
            off = 896 - 128 * j
            pltpu.async_copy(
                slabs[s].at[:, pl.ds(off, K)], dst_ref(g), osems[s]
            )

    def wait_outs(s):
        for _ in range(8):
            pltpu.make_async_copy(slabs[s], dst_ref(0), osems[s]).wait()

    fire_in(0, 0)
    fire_in(1, 1)
    for p in range(NPHASE):
        s = p % 3
        wait_in(s)
        fire_outs(p, s)
        if p + 2 < NPHASE:
            sn = (p + 2) % 3
            if p >= 1:
                wait_outs(sn)
            fire_in(p + 2, sn)
    for p in (NPHASE - 2, NPHASE - 1, NPHASE - 3):
        wait_outs(p % 3)


def kernel(query_length, key_length, table):
    shift = (jnp.asarray(key_length, jnp.int32) - K) - (
        jnp.asarray(query_length, jnp.int32) - Q
    )
    shift_arr = jnp.reshape(shift, (1,))

    stair = pl.pallas_call(
        _tc_stair_body,
        out_shape=jax.ShapeDtypeStruct((NPHASE, NUM_HEADS, 8, SROW), jnp.float32),
        in_specs=[
            pl.BlockSpec(memory_space=pltpu.SMEM),
            pl.BlockSpec(),
        ],
        compiler_params=pltpu.CompilerParams(
            vmem_limit_bytes=100 * 1024 * 1024,
        ),
    )(shift_arr, table)
    stair2d = stair.reshape(NPHASE * NUM_HEADS * 8, SROW)

    mesh = plsc.VectorSubcoreMesh(core_axis_name="c", subcore_axis_name="s")
    return pl.kernel(
        _sc_body,
        out_type=jax.ShapeDtypeStruct((1, NUM_HEADS, Q, K), jnp.float32),
        mesh=mesh,
        scratch_types=(
            [pltpu.VMEM((8, SLABW), jnp.float32)] * 3
            + [pltpu.SemaphoreType.DMA] * 6
        ),
        compiler_params=pltpu.CompilerParams(use_tc_tiling_on_sc=True),
    )(stair2d)


# speedup vs baseline: 2.4742x; 1.0358x over previous
import math

import jax
import jax.numpy as jnp
from jax import lax
from jax.experimental import pallas as pl
from jax.experimental.pallas import tpu as pltpu
from jax.experimental.pallas import tpu_sc as plsc

NUM_BUCKETS = 32
NUM_HEADS = 16
Q = 2048
K = 2048
NB = NUM_BUCKETS // 2
MAX_EXACT = NB // 2
MAX_DISTANCE = 128

TPAD = 4224
SROW = 4096
NPHASE = 16
NSLOT = 4
GROUPS = (Q // 2) // 8


def _tc_stair_body(shift_ref, table_ref, out_ref):
    shift = shift_ref[0]
    t = lax.broadcasted_iota(jnp.int32, (1, TPAD), 1)
    d = t - (Q - 1) + shift
    buckets = (d > 0).astype(jnp.int32) * NB
    rp = jnp.abs(d)
    is_small = rp < MAX_EXACT
    rp_safe = jnp.maximum(rp, 1)
    rp_if_large = MAX_EXACT + (
        jnp.log(rp_safe.astype(jnp.float32) / MAX_EXACT)
        / math.log(MAX_DISTANCE / MAX_EXACT)
        * (NB - MAX_EXACT)
    ).astype(jnp.int32)
    rp_if_large = jnp.minimum(rp_if_large, NB - 1)
    b = buckets + jnp.where(is_small, rp, rp_if_large)
    table = table_ref[...]
    w = jnp.zeros((NUM_HEADS, TPAD), jnp.float32)
    for v in range(NUM_BUCKETS):
        tcol = table[v, :].reshape(NUM_HEADS, 1)
        w = jnp.where(b == v, tcol, w)
    for p in range(NPHASE):
        for k in range(8):
            start = 8 * p + (7 - k)
            out_ref[p, :, k, :] = w[:, start:start + SROW]


SLABW = 2944
JSTRIDE = 16


def _sc_body(stair_hbm, out_hbm, b0, b1, b2, b3,
             si0, si1, si2, si3, so0, so1, so2, so3):
    head = lax.axis_index("s")
    half = lax.axis_index("c")
    slabs = (b0, b1, b2, b3)
    isems = (si0, si1, si2, si3)
    osems = (so0, so1, so2, so3)

    base_row = half * (Q // 2)
    slab_col = pl.multiple_of((Q // 2) - base_row, 128)

    def fire_in(p, s):
        row = pl.multiple_of((p * NUM_HEADS + head) * 8, 8)
        pltpu.async_copy(
            stair_hbm.at[pl.ds(row, 8), pl.ds(slab_col, SLABW)],
            slabs[s], isems[s],
        )

    def wait_in(s):
        pltpu.make_async_copy(
            stair_hbm.at[pl.ds(0, 8), pl.ds(0, SLABW)], slabs[s], isems[s]
        ).wait()

    def dst_ref(g):
        i = base_row + 8 * g
        return out_hbm.at[0, head, pl.ds(i, 8), :]

    def fire_outs(p, s):
        gp = (NPHASE - 1) - p
        for j in range(8):
            g = gp + JSTRIDE * j
            off = 896 - 128 * j
            pltpu.async_copy(
                slabs[s].at[:, pl.ds(off, K)], dst_ref(g), osems[s]
            )

    def wait_outs(s):
        for _ in range(8):
            pltpu.make_async_copy(slabs[s], dst_ref(0), osems[s]).wait()

    for p0 in range(3):
        fire_in(p0, p0)
    for p in range(NPHASE):
        s = p % 4
        wait_in(s)
        fire_outs(p, s)
        if p + 3 < NPHASE:
            sn = (p + 3) % 4
            if p >= 1:
                wait_outs(sn)
            fire_in(p + 3, sn)
    for p in range(NPHASE - 4, NPHASE):
        wait_outs(p % 4)


def kernel(query_length, key_length, table):
    shift = (jnp.asarray(key_length, jnp.int32) - K) - (
        jnp.asarray(query_length, jnp.int32) - Q
    )
    shift_arr = jnp.reshape(shift, (1,))

    stair = pl.pallas_call(
        _tc_stair_body,
        out_shape=jax.ShapeDtypeStruct((NPHASE, NUM_HEADS, 8, SROW), jnp.float32),
        in_specs=[
            pl.BlockSpec(memory_space=pltpu.SMEM),
            pl.BlockSpec(),
        ],
        compiler_params=pltpu.CompilerParams(
            vmem_limit_bytes=100 * 1024 * 1024,
        ),
    )(shift_arr, table)
    stair2d = stair.reshape(NPHASE * NUM_HEADS * 8, SROW)

    mesh = plsc.VectorSubcoreMesh(core_axis_name="c", subcore_axis_name="s")
    return pl.kernel(
        _sc_body,
        out_type=jax.ShapeDtypeStruct((1, NUM_HEADS, Q, K), jnp.float32),
        mesh=mesh,
        scratch_types=(
            [pltpu.VMEM((8, SLABW), jnp.float32)] * 4
            + [pltpu.SemaphoreType.DMA] * 8
        ),
        compiler_params=pltpu.CompilerParams(use_tc_tiling_on_sc=True),
    )(stair2d)


# speedup vs baseline: 2.5253x; 1.0206x over previous
import math

import jax
import jax.numpy as jnp
from jax import lax
from jax.experimental import pallas as pl
from jax.experimental.pallas import tpu as pltpu
from jax.experimental.pallas import tpu_sc as plsc

NUM_BUCKETS = 32
NUM_HEADS = 16
Q = 2048
K = 2048
NB = NUM_BUCKETS // 2
MAX_EXACT = NB // 2
MAX_DISTANCE = 128

TPAD = 4224
SROW = 4096
NPHASE = 16
NSLOT = 4
GROUPS = (Q // 2) // 8


def _tc_stair_body(shift_ref, table_ref, out_ref, stg0, stg1, sem0, sem1):
    shift = shift_ref[0]
    t = lax.broadcasted_iota(jnp.int32, (1, TPAD), 1)
    d = t - (Q - 1) + shift
    buckets = (d > 0).astype(jnp.int32) * NB
    rp = jnp.abs(d)
    is_small = rp < MAX_EXACT
    rp_safe = jnp.maximum(rp, 1)
    rp_if_large = MAX_EXACT + (
        jnp.log(rp_safe.astype(jnp.float32) / MAX_EXACT)
        / math.log(MAX_DISTANCE / MAX_EXACT)
        * (NB - MAX_EXACT)
    ).astype(jnp.int32)
    rp_if_large = jnp.minimum(rp_if_large, NB - 1)
    b = buckets + jnp.where(is_small, rp, rp_if_large)
    table = table_ref[...]
    w = jnp.zeros((NUM_HEADS, TPAD), jnp.float32)
    for v in range(NUM_BUCKETS):
        tcol = table[v, :].reshape(NUM_HEADS, 1)
        w = jnp.where(b == v, tcol, w)
    stgs, sems, cps = (stg0, stg1), (sem0, sem1), [None] * NPHASE
    for p in range(NPHASE):
        s = p % 2
        if p >= 2:
            cps[p - 2].wait()
        for k in range(8):
            start = 8 * p + (7 - k)
            stgs[s][:, k, :] = w[:, start:start + SROW]
        cps[p] = pltpu.async_copy(stgs[s], out_ref.at[p], sems[s])
    cps[NPHASE - 2].wait()
    cps[NPHASE - 1].wait()


SLABW = 2944
JSTRIDE = 16


def _sc_body(stair_hbm, out_hbm, b0, b1, b2, b3,
             si0, si1, si2, si3, so0, so1, so2, so3):
    head = lax.axis_index("s")
    half = lax.axis_index("c")
    slabs = (b0, b1, b2, b3)
    isems = (si0, si1, si2, si3)
    osems = (so0, so1, so2, so3)

    base_row = half * (Q // 2)
    slab_col = pl.multiple_of((Q // 2) - base_row, 128)

    def fire_in(p, s):
        row = pl.multiple_of((p * NUM_HEADS + head) * 8, 8)
        pltpu.async_copy(
            stair_hbm.at[pl.ds(row, 8), pl.ds(slab_col, SLABW)],
            slabs[s], isems[s],
        )

    def wait_in(s):
        pltpu.make_async_copy(
            stair_hbm.at[pl.ds(0, 8), pl.ds(0, SLABW)], slabs[s], isems[s]
        ).wait()

    def dst_ref(g):
        i = base_row + 8 * g
        return out_hbm.at[0, head, pl.ds(i, 8), :]

    def fire_outs(p, s):
        gp = (NPHASE - 1) - p
        for j in range(8):
            g = gp + JSTRIDE * j
            off = 896 - 128 * j
            pltpu.async_copy(
                slabs[s].at[:, pl.ds(off, K)], dst_ref(g), osems[s]
            )

    def wait_outs(s):
        for _ in range(8):
            pltpu.make_async_copy(slabs[s], dst_ref(0), osems[s]).wait()

    for p0 in range(3):
        fire_in(p0, p0)
    for p in range(NPHASE):
        s = p % 4
        wait_in(s)
        fire_outs(p, s)
        if p + 3 < NPHASE:
            sn = (p + 3) % 4
            if p >= 1:
                wait_outs(sn)
            fire_in(p + 3, sn)
    for p in range(NPHASE - 4, NPHASE):
        wait_outs(p % 4)


def kernel(query_length, key_length, table):
    shift = (jnp.asarray(key_length, jnp.int32) - K) - (
        jnp.asarray(query_length, jnp.int32) - Q
    )
    shift_arr = jnp.reshape(shift, (1,))

    stair = pl.pallas_call(
        _tc_stair_body,
        out_shape=jax.ShapeDtypeStruct((NPHASE, NUM_HEADS, 8, SROW), jnp.float32),
        in_specs=[
            pl.BlockSpec(memory_space=pltpu.SMEM),
            pl.BlockSpec(),
        ],
        out_specs=pl.BlockSpec(memory_space=pltpu.MemorySpace.HBM),
        scratch_shapes=[
            pltpu.VMEM((NUM_HEADS, 8, SROW), jnp.float32),
            pltpu.VMEM((NUM_HEADS, 8, SROW), jnp.float32),
            pltpu.SemaphoreType.DMA,
            pltpu.SemaphoreType.DMA,
        ],
        compiler_params=pltpu.CompilerParams(
            vmem_limit_bytes=100 * 1024 * 1024,
        ),
    )(shift_arr, table)
    stair2d = stair.reshape(NPHASE * NUM_HEADS * 8, SROW)

    mesh = plsc.VectorSubcoreMesh(core_axis_name="c", subcore_axis_name="s")
    return pl.kernel(
        _sc_body,
        out_type=jax.ShapeDtypeStruct((1, NUM_HEADS, Q, K), jnp.float32),
        mesh=mesh,
        scratch_types=(
            [pltpu.VMEM((8, SLABW), jnp.float32)] * 4
            + [pltpu.SemaphoreType.DMA] * 8
        ),
        compiler_params=pltpu.CompilerParams(use_tc_tiling_on_sc=True),
    )(stair2d)


# speedup vs baseline: 2.5667x; 1.0164x over previous
import math

import jax
import jax.numpy as jnp
from jax import lax
from jax.experimental import pallas as pl
from jax.experimental.pallas import tpu as pltpu
from jax.experimental.pallas import tpu_sc as plsc

NUM_BUCKETS = 32
NUM_HEADS = 16
Q = 2048
K = 2048
NB = NUM_BUCKETS // 2
MAX_EXACT = NB // 2
MAX_DISTANCE = 128

TPAD = 4224
SROW = 4096
NPHASE = 16
NSLOT = 4
GROUPS = (Q // 2) // 8


def _tc_stair_body(shift_ref, table_ref, out_ref, stg0, stg1, sem0, sem1):
    shift = shift_ref[0]
    t = lax.broadcasted_iota(jnp.int32, (1, TPAD), 1)
    d = t - (Q - 1) + shift
    buckets = (d > 0).astype(jnp.int32) * NB
    rp = jnp.abs(d)
    is_small = rp < MAX_EXACT
    rp_safe = jnp.maximum(rp, 1)
    rp_if_large = MAX_EXACT + (
        jnp.log(rp_safe.astype(jnp.float32) / MAX_EXACT)
        / math.log(MAX_DISTANCE / MAX_EXACT)
        * (NB - MAX_EXACT)
    ).astype(jnp.int32)
    rp_if_large = jnp.minimum(rp_if_large, NB - 1)
    b = buckets + jnp.where(is_small, rp, rp_if_large)
    table = table_ref[...]
    w = jnp.zeros((NUM_HEADS, TPAD), jnp.float32)
    for v in range(NUM_BUCKETS):
        tcol = table[v, :].reshape(NUM_HEADS, 1)
        w = jnp.where(b == v, tcol, w)
    stgs, sems, cps = (stg0, stg1), (sem0, sem1), [None] * NPHASE
    for p in range(NPHASE):
        s = p % 2
        if p >= 2:
            cps[p - 2].wait()
        for k in range(8):
            start = 8 * p + (7 - k)
            stgs[s][:, k, :] = w[:, start:start + SROW]
        cps[p] = pltpu.async_copy(stgs[s], out_ref.at[p], sems[s])
    cps[NPHASE - 2].wait()
    cps[NPHASE - 1].wait()


SLABW = 2944
JSTRIDE = 16


def _sc_body(stair_hbm, out_hbm, b0, b1, b2, b3, b4,
             si0, si1, si2, si3, si4, so0, so1, so2, so3, so4):
    head = lax.axis_index("s")
    half = lax.axis_index("c")
    slabs = (b0, b1, b2, b3, b4)
    isems = (si0, si1, si2, si3, si4)
    osems = (so0, so1, so2, so3, so4)

    base_row = half * (Q // 2)
    slab_col = pl.multiple_of((Q // 2) - base_row, 128)

    def fire_in(p, s):
        row = pl.multiple_of((p * NUM_HEADS + head) * 8, 8)
        pltpu.async_copy(
            stair_hbm.at[pl.ds(row, 8), pl.ds(slab_col, SLABW)],
            slabs[s], isems[s],
        )

    def wait_in(s):
        pltpu.make_async_copy(
            stair_hbm.at[pl.ds(0, 8), pl.ds(0, SLABW)], slabs[s], isems[s]
        ).wait()

    def dst_ref(g):
        i = base_row + 8 * g
        return out_hbm.at[0, head, pl.ds(i, 8), :]

    def fire_outs(p, s):
        gp = (NPHASE - 1) - p
        for j in range(8):
            g = gp + JSTRIDE * j
            off = 896 - 128 * j
            pltpu.async_copy(
                slabs[s].at[:, pl.ds(off, K)], dst_ref(g), osems[s]
            )

    def wait_outs(s):
        for _ in range(8):
            pltpu.make_async_copy(slabs[s], dst_ref(0), osems[s]).wait()

    for p0 in range(4):
        fire_in(p0, p0)
    for p in range(NPHASE):
        s = p % 5
        wait_in(s)
        fire_outs(p, s)
        if p + 4 < NPHASE:
            sn = (p + 4) % 5
            if p >= 1:
                wait_outs(sn)
            fire_in(p + 4, sn)
    for p in range(NPHASE - 5, NPHASE):
        wait_outs(p % 5)


def kernel(query_length, key_length, table):
    shift = (jnp.asarray(key_length, jnp.int32) - K) - (
        jnp.asarray(query_length, jnp.int32) - Q
    )
    shift_arr = jnp.reshape(shift, (1,))

    stair = pl.pallas_call(
        _tc_stair_body,
        out_shape=jax.ShapeDtypeStruct((NPHASE, NUM_HEADS, 8, SROW), jnp.float32),
        in_specs=[
            pl.BlockSpec(memory_space=pltpu.SMEM),
            pl.BlockSpec(),
        ],
        out_specs=pl.BlockSpec(memory_space=pltpu.MemorySpace.HBM),
        scratch_shapes=[
            pltpu.VMEM((NUM_HEADS, 8, SROW), jnp.float32),
            pltpu.VMEM((NUM_HEADS, 8, SROW), jnp.float32),
            pltpu.SemaphoreType.DMA,
            pltpu.SemaphoreType.DMA,
        ],
        compiler_params=pltpu.CompilerParams(
            vmem_limit_bytes=100 * 1024 * 1024,
        ),
    )(shift_arr, table)
    stair2d = stair.reshape(NPHASE * NUM_HEADS * 8, SROW)

    mesh = plsc.VectorSubcoreMesh(core_axis_name="c", subcore_axis_name="s")
    return pl.kernel(
        _sc_body,
        out_type=jax.ShapeDtypeStruct((1, NUM_HEADS, Q, K), jnp.float32),
        mesh=mesh,
        scratch_types=(
            [pltpu.VMEM((8, SLABW), jnp.float32)] * 5
            + [pltpu.SemaphoreType.DMA] * 10
        ),
        compiler_params=pltpu.CompilerParams(use_tc_tiling_on_sc=True),
    )(stair2d)


# speedup vs baseline: 2.7617x; 1.0760x over previous
import math

import jax
import jax.numpy as jnp
from jax import lax
from jax.experimental import pallas as pl
from jax.experimental.pallas import tpu as pltpu
from jax.experimental.pallas import tpu_sc as plsc

NUM_BUCKETS = 32
NUM_HEADS = 16
Q = 2048
K = 2048
NB = NUM_BUCKETS // 2
MAX_EXACT = NB // 2
MAX_DISTANCE = 128

TPAD = 4224
SROW = 4096
NPHASE = 16
NSLOT = 4
GROUPS = (Q // 2) // 8


def _tc_stair_body(shift_ref, table_ref, out_ref, stg0, stg1, sem0, sem1):
    shift = shift_ref[0]
    t = lax.broadcasted_iota(jnp.int32, (1, TPAD), 1)
    d = t - (Q - 1) + shift
    buckets = (d > 0).astype(jnp.int32) * NB
    rp = jnp.abs(d)
    is_small = rp < MAX_EXACT
    rp_safe = jnp.maximum(rp, 1)
    rp_if_large = MAX_EXACT + (
        jnp.log(rp_safe.astype(jnp.float32) / MAX_EXACT)
        / math.log(MAX_DISTANCE / MAX_EXACT)
        * (NB - MAX_EXACT)
    ).astype(jnp.int32)
    rp_if_large = jnp.minimum(rp_if_large, NB - 1)
    b = buckets + jnp.where(is_small, rp, rp_if_large)
    table = table_ref[...]
    w = jnp.zeros((NUM_HEADS, TPAD), jnp.float32)
    for v in range(NUM_BUCKETS):
        tcol = table[v, :].reshape(NUM_HEADS, 1)
        w = jnp.where(b == v, tcol, w)
    stgs, sems, cps = (stg0, stg1), (sem0, sem1), [None] * NPHASE
    for p in range(NPHASE):
        s = p % 2
        if p >= 2:
            cps[p - 2].wait()
        for k in range(8):
            start = 8 * p + (7 - k)
            stgs[s][:, k, :] = w[:, start:start + SROW]
        cps[p] = pltpu.async_copy(stgs[s], out_ref.at[p], sems[s])
    cps[NPHASE - 2].wait()
    cps[NPHASE - 1].wait()


SLABW = 3968
NPH_SUB = NPHASE // 2


def _sc_body(stair_hbm, out_hbm, b0, b1, b2, b3,
             si0, si1, si2, si3, so0, so1, so2, so3):
    head = lax.axis_index("s")
    phalf = lax.axis_index("c")
    slabs = (b0, b1, b2, b3)
    isems = (si0, si1, si2, si3)
    osems = (so0, so1, so2, so3)

    pbase = phalf * NPH_SUB

    def fire_in(q, s):
        row = pl.multiple_of(((pbase + q) * NUM_HEADS + head) * 8, 8)
        pltpu.async_copy(
            stair_hbm.at[pl.ds(row, 8), pl.ds(0, SLABW)],
            slabs[s], isems[s],
        )

    def wait_in(s):
        pltpu.make_async_copy(
            stair_hbm.at[pl.ds(0, 8), pl.ds(0, SLABW)], slabs[s], isems[s]
        ).wait()

    def dst_ref(i):
        return out_hbm.at[0, head, pl.ds(i, 8), :]

    def fire_outs(q, s):
        for j in range(NPHASE):
            i = 8 * (NPHASE - 1) + 128 * j - 8 * (pbase + q)
            off = (NPHASE - 1) * 128 - 128 * j
            pltpu.async_copy(
                slabs[s].at[:, pl.ds(off, K)], dst_ref(i), osems[s]
            )

    def wait_outs(s):
        for _ in range(NPHASE):
            pltpu.make_async_copy(slabs[s], dst_ref(0), osems[s]).wait()

    for q0 in range(3):
        fire_in(q0, q0)
    for q in range(NPH_SUB):
        s = q % 4
        wait_in(s)
        fire_outs(q, s)
        if q + 3 < NPH_SUB:
            sn = (q + 3) % 4
            if q >= 1:
                wait_outs(sn)
            fire_in(q + 3, sn)
    for q in range(NPH_SUB - 4, NPH_SUB):
        wait_outs(q % 4)


def kernel(query_length, key_length, table):
    shift = (jnp.asarray(key_length, jnp.int32) - K) - (
        jnp.asarray(query_length, jnp.int32) - Q
    )
    shift_arr = jnp.reshape(shift, (1,))

    stair = pl.pallas_call(
        _tc_stair_body,
        out_shape=jax.ShapeDtypeStruct((NPHASE, NUM_HEADS, 8, SROW), jnp.float32),
        in_specs=[
            pl.BlockSpec(memory_space=pltpu.SMEM),
            pl.BlockSpec(),
        ],
        out_specs=pl.BlockSpec(memory_space=pltpu.MemorySpace.HBM),
        scratch_shapes=[
            pltpu.VMEM((NUM_HEADS, 8, SROW), jnp.float32),
            pltpu.VMEM((NUM_HEADS, 8, SROW), jnp.float32),
            pltpu.SemaphoreType.DMA,
            pltpu.SemaphoreType.DMA,
        ],
        compiler_params=pltpu.CompilerParams(
            vmem_limit_bytes=100 * 1024 * 1024,
        ),
    )(shift_arr, table)
    stair2d = stair.reshape(NPHASE * NUM_HEADS * 8, SROW)

    mesh = plsc.VectorSubcoreMesh(core_axis_name="c", subcore_axis_name="s")
    return pl.kernel(
        _sc_body,
        out_type=jax.ShapeDtypeStruct((1, NUM_HEADS, Q, K), jnp.float32),
        mesh=mesh,
        scratch_types=(
            [pltpu.VMEM((8, SLABW), jnp.float32)] * 4
            + [pltpu.SemaphoreType.DMA] * 8
        ),
        compiler_params=pltpu.CompilerParams(use_tc_tiling_on_sc=True),
    )(stair2d)


# speedup vs baseline: 2.8358x; 1.0268x over previous
import math

import jax
import jax.numpy as jnp
from jax import lax
from jax.experimental import pallas as pl
from jax.experimental.pallas import tpu as pltpu
from jax.experimental.pallas import tpu_sc as plsc

NUM_BUCKETS = 32
NUM_HEADS = 16
Q = 2048
K = 2048
NB = NUM_BUCKETS // 2
MAX_EXACT = NB // 2
MAX_DISTANCE = 128

TPAD = 4224
SROW = 4096
NPHASE = 16
NSLOT = 4
GROUPS = (Q // 2) // 8


def _tc_stair_body(shift_ref, table_ref, out_ref, stg0, stg1, stg2, stg3,
                   sem0, sem1, sem2, sem3):
    shift = shift_ref[0]
    t = lax.broadcasted_iota(jnp.int32, (1, TPAD), 1)
    d = t - (Q - 1) + shift
    buckets = (d > 0).astype(jnp.int32) * NB
    rp = jnp.abs(d)
    is_small = rp < MAX_EXACT
    rp_safe = jnp.maximum(rp, 1)
    rp_if_large = MAX_EXACT + (
        jnp.log(rp_safe.astype(jnp.float32) / MAX_EXACT)
        / math.log(MAX_DISTANCE / MAX_EXACT)
        * (NB - MAX_EXACT)
    ).astype(jnp.int32)
    rp_if_large = jnp.minimum(rp_if_large, NB - 1)
    b = buckets + jnp.where(is_small, rp, rp_if_large)
    table = table_ref[...]
    w = jnp.zeros((NUM_HEADS, TPAD), jnp.float32)
    for v in range(NUM_BUCKETS):
        tcol = table[v, :].reshape(NUM_HEADS, 1)
        w = jnp.where(b == v, tcol, w)
    stgs = (stg0, stg1, stg2, stg3)
    sems = (sem0, sem1, sem2, sem3)
    cps = [None] * NPHASE
    for p in range(NPHASE):
        s = p % 4
        if p >= 4:
            cps[p - 4].wait()
        for k in range(8):
            start = 8 * p + (7 - k)
            stgs[s][:, k, :] = w[:, start:start + SROW]
        cps[p] = pltpu.async_copy(stgs[s], out_ref.at[p], sems[s])
    for p in range(NPHASE - 4, NPHASE):
        cps[p].wait()


SLABW = 3968
NPH_SUB = NPHASE // 2


def _sc_body(stair_hbm, out_hbm, b0, b1, b2, b3,
             si0, si1, si2, si3, so0, so1, so2, so3):
    head = lax.axis_index("s")
    phalf = lax.axis_index("c")
    slabs = (b0, b1, b2, b3)
    isems = (si0, si1, si2, si3)
    osems = (so0, so1, so2, so3)

    pbase = phalf * NPH_SUB

    def fire_in(q, s):
        row = pl.multiple_of(((pbase + q) * NUM_HEADS + head) * 8, 8)
        pltpu.async_copy(
            stair_hbm.at[pl.ds(row, 8), pl.ds(0, SLABW)],
            slabs[s], isems[s],
        )

    def wait_in(s):
        pltpu.make_async_copy(
            stair_hbm.at[pl.ds(0, 8), pl.ds(0, SLABW)], slabs[s], isems[s]
        ).wait()

    def dst_ref(i):
        return out_hbm.at[0, head, pl.ds(i, 8), :]

    def fire_outs(q, s):
        for j in range(NPHASE):
            i = 8 * (NPHASE - 1) + 128 * j - 8 * (pbase + q)
            off = (NPHASE - 1) * 128 - 128 * j
            pltpu.async_copy(
                slabs[s].at[:, pl.ds(off, K)], dst_ref(i), osems[s]
            )

    def wait_outs(s):
        for _ in range(NPHASE):
            pltpu.make_async_copy(slabs[s], dst_ref(0), osems[s]).wait()

    for q0 in range(3):
        fire_in(q0, q0)
    for q in range(NPH_SUB):
        s = q % 4
        wait_in(s)
        fire_outs(q, s)
        if q + 3 < NPH_SUB:
            sn = (q + 3) % 4
            if q >= 1:
                wait_outs(sn)
            fire_in(q + 3, sn)
    for q in range(NPH_SUB - 4, NPH_SUB):
        wait_outs(q % 4)


def kernel(query_length, key_length, table):
    shift = (jnp.asarray(key_length, jnp.int32) - K) - (
        jnp.asarray(query_length, jnp.int32) - Q
    )
    shift_arr = jnp.reshape(shift, (1,))

    stair = pl.pallas_call(
        _tc_stair_body,
        out_shape=jax.ShapeDtypeStruct((NPHASE, NUM_HEADS, 8, SROW), jnp.float32),
        in_specs=[
            pl.BlockSpec(memory_space=pltpu.SMEM),
            pl.BlockSpec(),
        ],
        out_specs=pl.BlockSpec(memory_space=pltpu.MemorySpace.HBM),
        scratch_shapes=(
            [pltpu.VMEM((NUM_HEADS, 8, SROW), jnp.float32)] * 4
            + [pltpu.SemaphoreType.DMA] * 4
        ),
        compiler_params=pltpu.CompilerParams(
            vmem_limit_bytes=100 * 1024 * 1024,
        ),
    )(shift_arr, table)
    stair2d = stair.reshape(NPHASE * NUM_HEADS * 8, SROW)

    mesh = plsc.VectorSubcoreMesh(core_axis_name="c", subcore_axis_name="s")
    return pl.kernel(
        _sc_body,
        out_type=jax.ShapeDtypeStruct((1, NUM_HEADS, Q, K), jnp.float32),
        mesh=mesh,
        scratch_types=(
            [pltpu.VMEM((8, SLABW), jnp.float32)] * 4
            + [pltpu.SemaphoreType.DMA] * 8
        ),
        compiler_params=pltpu.CompilerParams(use_tc_tiling_on_sc=True),
    )(stair2d)


# speedup vs baseline: 2.8474x; 1.0041x over previous
import math

import jax
import jax.numpy as jnp
from jax import lax
from jax.experimental import pallas as pl
from jax.experimental.pallas import tpu as pltpu
from jax.experimental.pallas import tpu_sc as plsc

NUM_BUCKETS = 32
NUM_HEADS = 16
Q = 2048
K = 2048
NB = NUM_BUCKETS // 2
MAX_EXACT = NB // 2
MAX_DISTANCE = 128

TPAD = 4224
SROW = 4096
NPHASE = 16


def _tc_stair_body(shift_ref, table_ref, out_ref, stg0, stg1, stg2, stg3,
                   sem0, sem1, sem2, sem3):
    shift = shift_ref[0]
    t = lax.broadcasted_iota(jnp.int32, (1, TPAD), 1)
    d = t - (Q - 1) + shift
    buckets = (d > 0).astype(jnp.int32) * NB
    rp = jnp.abs(d)
    is_small = rp < MAX_EXACT
    rp_safe = jnp.maximum(rp, 1)
    rp_if_large = MAX_EXACT + (
        jnp.log(rp_safe.astype(jnp.float32) / MAX_EXACT)
        / math.log(MAX_DISTANCE / MAX_EXACT)
        * (NB - MAX_EXACT)
    ).astype(jnp.int32)
    rp_if_large = jnp.minimum(rp_if_large, NB - 1)
    b = buckets + jnp.where(is_small, rp, rp_if_large)
    table = table_ref[...]
    w = jnp.zeros((NUM_HEADS, TPAD), jnp.float32)
    for v in range(NUM_BUCKETS):
        tcol = table[v, :].reshape(NUM_HEADS, 1)
        w = jnp.where(b == v, tcol, w)
    stgs = (stg0, stg1, stg2, stg3)
    sems = (sem0, sem1, sem2, sem3)
    cps = [None] * NPHASE
    for p in range(NPHASE):
        s = p % 4
        if p >= 4:
            cps[p - 4].wait()
        for k in range(8):
            start = 8 * p + (7 - k)
            stgs[s][:, k, :] = w[:, start:start + SROW]
        cps[p] = pltpu.async_copy(stgs[s], out_ref.at[p], sems[s])
    for p in range(NPHASE - 4, NPHASE):
        cps[p].wait()


SLABW = 3968
NPH_SUB = NPHASE // 2


def _sc_body(stair_hbm, out_hbm, b0, b1, b2, b3,
             si0, si1, si2, si3, so0, so1, so2, so3):
    head = lax.axis_index("s")
    phalf = lax.axis_index("c")
    slabs = (b0, b1, b2, b3)
    isems = (si0, si1, si2, si3)
    osems = (so0, so1, so2, so3)

    pbase = phalf * NPH_SUB

    def fire_in(q, s):
        row = pl.multiple_of(((pbase + q) * NUM_HEADS + head) * 8, 8)
        pltpu.async_copy(
            stair_hbm.at[pl.ds(row, 8), pl.ds(0, SLABW)],
            slabs[s], isems[s],
        )

    def wait_in(s):
        pltpu.make_async_copy(
            stair_hbm.at[pl.ds(0, 8), pl.ds(0, SLABW)], slabs[s], isems[s]
        ).wait()

    def dst_ref(i):
        return out_hbm.at[0, head, pl.ds(i, 8), :]

    def fire_outs(q, s):
        for j in range(NPHASE):
            i = 8 * (NPHASE - 1) + 128 * j - 8 * (pbase + q)
            off = (NPHASE - 1) * 128 - 128 * j
            pltpu.async_copy(
                slabs[s].at[:, pl.ds(off, K)], dst_ref(i), osems[s]
            )

    def wait_outs(s):
        for _ in range(NPHASE):
            pltpu.make_async_copy(slabs[s], dst_ref(0), osems[s]).wait()

    for q0 in range(3):
        fire_in(q0, q0)
    for q in range(NPH_SUB):
        s = q % 4
        wait_in(s)
        fire_outs(q, s)
        if q + 3 < NPH_SUB:
            sn = (q + 3) % 4
            if q >= 1:
                wait_outs(sn)
            fire_in(q + 3, sn)
    for q in range(NPH_SUB - 4, NPH_SUB):
        wait_outs(q % 4)


def kernel(query_length, key_length, table):
    shift = (jnp.asarray(key_length, jnp.int32) - K) - (
        jnp.asarray(query_length, jnp.int32) - Q
    )
    shift_arr = jnp.reshape(shift, (1,))

    stair = pl.pallas_call(
        _tc_stair_body,
        out_shape=jax.ShapeDtypeStruct((NPHASE, NUM_HEADS, 8, SROW), jnp.float32),
        in_specs=[
            pl.BlockSpec(memory_space=pltpu.SMEM),
            pl.BlockSpec(),
        ],
        out_specs=pl.BlockSpec(memory_space=pltpu.MemorySpace.HBM),
        scratch_shapes=(
            [pltpu.VMEM((NUM_HEADS, 8, SROW), jnp.float32)] * 4
            + [pltpu.SemaphoreType.DMA] * 4
        ),
        compiler_params=pltpu.CompilerParams(
            vmem_limit_bytes=100 * 1024 * 1024,
        ),
    )(shift_arr, table)
    stair2d = stair.reshape(NPHASE * NUM_HEADS * 8, SROW)

    mesh = plsc.VectorSubcoreMesh(core_axis_name="c", subcore_axis_name="s")
    return pl.kernel(
        _sc_body,
        out_type=jax.ShapeDtypeStruct((1, NUM_HEADS, Q, K), jnp.float32),
        mesh=mesh,
        scratch_types=(
            [pltpu.VMEM((8, SLABW), jnp.float32)] * 4
            + [pltpu.SemaphoreType.DMA] * 8
        ),
        compiler_params=pltpu.CompilerParams(use_tc_tiling_on_sc=True),
    )(stair2d)
